# Initial kernel scaffold; baseline (speedup 1.0000x reference)
import jax
import jax.numpy as jnp
from jax.experimental import pallas as pl


def kernel(x, edge_index, edge_attr, emb, W0, root0, bias0, gamma0, beta0, W1, root1, bias1, gamma1, beta1, finW, finb):
    raise NotImplementedError("write your pallas kernel here")


# speedup vs baseline: 1.7346x; 1.7346x over previous
import functools

import jax
import jax.numpy as jnp
from jax import lax
from jax.experimental import pallas as pl
from jax.experimental.pallas import tpu as pltpu
from jax.experimental.pallas import tpu_sc as plsc

N = 10000
E = 320000
C = 128
K = 5
KTOT = K * K

NC = 2
NS = 16
NW = NC * NS
EPW = E // NW
ESUB = E // NS
BB = 80
NB = ESUB // BB
HALF = N // NC
HROWS = 5120
NCH = HROWS // BB

_mesh = plsc.VectorSubcoreMesh(core_axis_name="c", subcore_axis_name="s")


def _prep_body(a0_ref, a1_ref, f0_ref, f1_ref, base_ref):
    p0 = a0_ref[...] * (K - 1.0)
    p1 = a1_ref[...] * (K - 1.0)
    lo0 = jnp.clip(jnp.floor(p0), 0.0, K - 2.0)
    lo1 = jnp.clip(jnp.floor(p1), 0.0, K - 2.0)
    f0_ref[...] = p0 - lo0
    f1_ref[...] = p1 - lo1
    base_ref[...] = lo0.astype(jnp.int32) + K * lo1.astype(jnp.int32)


def _edge_prep(a0, a1):
    return pl.pallas_call(
        _prep_body,
        out_shape=[
            jax.ShapeDtypeStruct(a0.shape, jnp.float32),
            jax.ShapeDtypeStruct(a0.shape, jnp.float32),
            jax.ShapeDtypeStruct(a0.shape, jnp.int32),
        ],
    )(a0, a1)


def _matmul_body(a_ref, b_ref, o_ref):
    o_ref[...] = jnp.dot(a_ref[...], b_ref[...],
                         preferred_element_type=jnp.float32)


def _matmul(a, b):
    return pl.pallas_call(
        _matmul_body,
        out_shape=jax.ShapeDtypeStruct((a.shape[0], b.shape[1]), jnp.float32),
    )(a, b)


def _p1_body(h_ref, w_ref, o_ref):
    o_ref[...] = jnp.dot(h_ref[...], w_ref[...],
                         preferred_element_type=jnp.float32)


def _p1_matmul(h, wt):
    rb = 1000
    return pl.pallas_call(
        _p1_body,
        grid=(N // rb,),
        in_specs=[
            pl.BlockSpec((rb, C), lambda i: (i, 0)),
            pl.BlockSpec((C, KTOT * C), lambda i: (0, 0)),
        ],
        out_specs=pl.BlockSpec((rb, KTOT * C), lambda i: (i, 0)),
        out_shape=jax.ShapeDtypeStruct((N, KTOT * C), jnp.float32),
    )(h, wt)


@functools.partial(
    pl.kernel,
    out_type=[
        jax.ShapeDtypeStruct((NC, HROWS, C), jnp.float32),
        jax.ShapeDtypeStruct((NW * N,), jnp.float32),
    ],
    mesh=_mesh,
    scratch_types=[
        pltpu.VMEM_SHARED((HROWS, C), jnp.float32),
        pltpu.VMEM((2 * KTOT, C), jnp.float32),
        pltpu.VMEM((BB, C), jnp.float32),
        pltpu.VMEM((N,), jnp.int32),
        pltpu.VMEM((N,), jnp.float32),
        pltpu.VMEM((BB,), jnp.int32),
        pltpu.VMEM((BB,), jnp.int32),
        pltpu.VMEM((BB,), jnp.int32),
        pltpu.VMEM((BB,), jnp.float32),
        pltpu.VMEM((BB,), jnp.float32),
        pltpu.VMEM((BB, C), jnp.float32),
    ],
)
def _l0_kernel(xb2_hbm, src_hbm, dst_hbm, f0_hbm, f1_hbm, base_hbm, p0_hbm,
               msg_out, deg_out,
               msg_sh, p0_v, z_v, xb_v, deg_v,
               src_v, dst_v, base_v, f0_v, f1_v, rows_v):
    cid = lax.axis_index("c")
    sid = lax.axis_index("s")
    wid = cid * NS + sid

    zero16 = jnp.zeros((16,), jnp.float32)
    iota16 = lax.iota(jnp.int32, 16)

    def zrow(i, carry):
        for c8 in range(C // 16):
            z_v[i, pl.ds(c8 * 16, 16)] = zero16
        return carry
    lax.fori_loop(0, BB, zrow, 0)

    def zdeg(i, carry):
        deg_v[pl.ds(i * 16, 16)] = zero16
        return carry
    lax.fori_loop(0, N // 16, zdeg, 0)

    for i in range(NCH // NS):
        ch = i * NS + sid
        pltpu.sync_copy(z_v, msg_sh.at[pl.ds(ch * BB, BB)])
    pltpu.sync_copy(p0_hbm, p0_v)
    pltpu.sync_copy(xb2_hbm, xb_v)
    plsc.subcore_barrier()

    def batch(it, carry):
        bbase = sid * ESUB + it * BB
        pltpu.sync_copy(src_hbm.at[pl.ds(bbase, BB)], src_v)
        pltpu.sync_copy(dst_hbm.at[pl.ds(bbase, BB)], dst_v)
        pltpu.sync_copy(f0_hbm.at[pl.ds(bbase, BB)], f0_v)
        pltpu.sync_copy(f1_hbm.at[pl.ds(bbase, BB)], f1_v)
        pltpu.sync_copy(base_hbm.at[pl.ds(bbase, BB)], base_v)

        def group(j, c2):
            sl = pl.ds(j * 16, 16)
            src16 = src_v[sl]
            dst16 = dst_v[sl]
            base16 = base_v[sl]
            f016 = f0_v[sl]
            f116 = f1_v[sl]
            rel16 = dst16 - cid * HALF
            ok16 = jnp.logical_and(rel16 >= 0, rel16 < HALF)
            dst_v[sl] = jnp.where(ok16, rel16, HALF)
            for lane in range(16):
                i = j * 16 + lane
                s = src16[lane]
                xw = xb_v[pl.ds((s >> 4) * 16, 16)]
                bit = (xw[0] >> (s & 15)) & 1
                g = bit * KTOT + base16[lane]
                d = dst16[lane]
                drow = (d >> 4) * 16
                dv = deg_v[pl.ds(drow, 16)]
                oh = jnp.where(iota16 == (d & 15), 1.0, 0.0)
                deg_v[pl.ds(drow, 16)] = dv + oh
                a0 = f016[lane]
                a1 = f116[lane]
                for c8 in range(C // 16):
                    cs = pl.ds(c8 * 16, 16)
                    va = p0_v[g, cs]
                    vb = p0_v[g + 1, cs]
                    vc = p0_v[g + K, cs]
                    vd = p0_v[g + K + 1, cs]
                    t0 = va + a0 * (vb - va)
                    t1 = vc + a0 * (vd - vc)
                    rows_v[i, cs] = t0 + a1 * (t1 - t0)
            return c2
        lax.fori_loop(0, BB // 16, group, 0)

        pltpu.sync_copy(rows_v, msg_sh.at[dst_v], add=True)
        return carry
    lax.fori_loop(0, NB, batch, 0)

    plsc.subcore_barrier()
    for i in range(NCH // NS):
        ch = i * NS + sid
        pltpu.sync_copy(msg_sh.at[pl.ds(ch * BB, BB)],
                        msg_out.at[cid, pl.ds(ch * BB, BB)])
    pltpu.sync_copy(deg_v, deg_out.at[pl.ds(wid * N, N)])


@functools.partial(
    pl.kernel,
    out_type=jax.ShapeDtypeStruct((NC, HROWS, C), jnp.float32),
    mesh=_mesh,
    scratch_types=[
        pltpu.VMEM_SHARED((HROWS, C), jnp.float32),
        pltpu.VMEM((BB, C), jnp.float32),
        pltpu.VMEM((BB,), jnp.int32),
        pltpu.VMEM((BB,), jnp.int32),
        pltpu.VMEM((BB,), jnp.int32),
        pltpu.VMEM((BB,), jnp.float32),
        pltpu.VMEM((BB,), jnp.float32),
        pltpu.VMEM((BB,), jnp.int32),
        pltpu.VMEM((BB,), jnp.int32),
        pltpu.VMEM((BB,), jnp.int32),
        pltpu.VMEM((BB,), jnp.int32),
        pltpu.VMEM((BB, C), jnp.float32),
        pltpu.VMEM((BB, C), jnp.float32),
        pltpu.VMEM((BB, C), jnp.float32),
        pltpu.VMEM((BB, C), jnp.float32),
        pltpu.VMEM((BB, C), jnp.float32),
        pltpu.SemaphoreType.DMA,
    ],
)
def _l1_kernel(src_hbm, dst_hbm, f0_hbm, f1_hbm, base_hbm, p1_hbm,
               msg_out,
               msg_sh, z_v, src_v, dst_v, base_v, f0_v, f1_v,
               ga_v, gb_v, gc_v, gd_v,
               ra_v, rb_v, rc_v, rd_v, rows_v, sem):
    cid = lax.axis_index("c")
    sid = lax.axis_index("s")
    wid = cid * NS + sid

    zero16 = jnp.zeros((16,), jnp.float32)

    def zrow(i, carry):
        for c8 in range(C // 16):
            z_v[i, pl.ds(c8 * 16, 16)] = zero16
        return carry
    lax.fori_loop(0, BB, zrow, 0)

    for i in range(NCH // NS):
        ch = i * NS + sid
        pltpu.sync_copy(z_v, msg_sh.at[pl.ds(ch * BB, BB)])
    plsc.subcore_barrier()

    def batch(it, carry):
        bbase = sid * ESUB + it * BB
        pltpu.sync_copy(src_hbm.at[pl.ds(bbase, BB)], src_v)
        pltpu.sync_copy(dst_hbm.at[pl.ds(bbase, BB)], dst_v)
        pltpu.sync_copy(f0_hbm.at[pl.ds(bbase, BB)], f0_v)
        pltpu.sync_copy(f1_hbm.at[pl.ds(bbase, BB)], f1_v)
        pltpu.sync_copy(base_hbm.at[pl.ds(bbase, BB)], base_v)

        for j in range(BB // 16):
            sl = pl.ds(j * 16, 16)
            g = src_v[sl] * KTOT + base_v[sl]
            ga_v[sl] = g
            gb_v[sl] = g + 1
            gc_v[sl] = g + K
            gd_v[sl] = g + K + 1
            rel16 = dst_v[sl] - cid * HALF
            ok16 = jnp.logical_and(rel16 >= 0, rel16 < HALF)
            dst_v[sl] = jnp.where(ok16, rel16, HALF)

        da = pltpu.async_copy(p1_hbm.at[ga_v], ra_v, sem)
        db = pltpu.async_copy(p1_hbm.at[gb_v], rb_v, sem)
        dc = pltpu.async_copy(p1_hbm.at[gc_v], rc_v, sem)
        dd = pltpu.async_copy(p1_hbm.at[gd_v], rd_v, sem)
        da.wait()
        db.wait()
        dc.wait()
        dd.wait()

        def group(j, c2):
            sl = pl.ds(j * 16, 16)
            f016 = f0_v[sl]
            f116 = f1_v[sl]
            for lane in range(16):
                i = j * 16 + lane
                a0 = f016[lane]
                a1 = f116[lane]
                for c8 in range(C // 16):
                    cs = pl.ds(c8 * 16, 16)
                    va = ra_v[i, cs]
                    vb = rb_v[i, cs]
                    vc = rc_v[i, cs]
                    vd = rd_v[i, cs]
                    t0 = va + a0 * (vb - va)
                    t1 = vc + a0 * (vd - vc)
                    rows_v[i, cs] = t0 + a1 * (t1 - t0)
            return c2
        lax.fori_loop(0, BB // 16, group, 0)

        pltpu.sync_copy(rows_v, msg_sh.at[dst_v], add=True)
        return carry
    lax.fori_loop(0, NB, batch, 0)

    plsc.subcore_barrier()
    for i in range(NCH // NS):
        ch = i * NS + sid
        pltpu.sync_copy(msg_sh.at[pl.ds(ch * BB, BB)],
                        msg_out.at[cid, pl.ds(ch * BB, BB)])


def _l0_dense_body(msgp_ref, degp_ref, x1_ref, emb_ref, root_ref,
                   bias_ref, gamma_ref, beta_ref, ones_w_ref, ones_1_ref,
                   h_ref):
    m = jnp.concatenate([msgp_ref[0, :HALF], msgp_ref[1, :HALF]], axis=0)
    degmat = lax.dot_general(degp_ref[...], ones_w_ref[...],
                             (((0,), (0,)), ((), ())),
                             preferred_element_type=jnp.float32)
    degmat = jnp.maximum(degmat, 1.0)
    er = jnp.dot(emb_ref[...], root_ref[...],
                 preferred_element_type=jnp.float32)
    xmat = lax.dot_general(x1_ref[...], ones_1_ref[...],
                           (((0,), (0,)), ((), ())),
                           preferred_element_type=jnp.float32)
    er0 = er[0:1, :]
    er1 = er[1:2, :]
    sel = er0 + xmat * (er1 - er0)
    msg = m / degmat + sel + bias_ref[...]
    mu = jnp.mean(msg, axis=0, keepdims=True)
    ctr = msg - mu
    var = jnp.mean(ctr * ctr, axis=0, keepdims=True)
    h = ctr * lax.rsqrt(var + 1e-5) * gamma_ref[...] + beta_ref[...]
    h_ref[...] = jnp.maximum(h, 0.0)


def _l0_dense(msgp, degp, x1, emb, root, bias, gamma, beta, ones_w, ones_1):
    return pl.pallas_call(
        _l0_dense_body,
        out_shape=jax.ShapeDtypeStruct((N, C), jnp.float32),
    )(msgp, degp, x1, emb, root, bias, gamma, beta, ones_w, ones_1)


def _l1_dense_body(msgp_ref, degp_ref, h1_ref, root_ref, bias_ref,
                   gamma_ref, beta_ref, finw_ref, finb_ref, ones_w_ref,
                   o_ref):
    m = jnp.concatenate([msgp_ref[0, :HALF], msgp_ref[1, :HALF]], axis=0)
    degmat = lax.dot_general(degp_ref[...], ones_w_ref[...],
                             (((0,), (0,)), ((), ())),
                             preferred_element_type=jnp.float32)
    degmat = jnp.maximum(degmat, 1.0)
    msg = m / degmat + jnp.dot(h1_ref[...], root_ref[...],
                               preferred_element_type=jnp.float32)
    msg = msg + bias_ref[...]
    mu = jnp.mean(msg, axis=0, keepdims=True)
    ctr = msg - mu
    var = jnp.mean(ctr * ctr, axis=0, keepdims=True)
    y = ctr * lax.rsqrt(var + 1e-5) * gamma_ref[...] + beta_ref[...]
    o_ref[...] = jnp.dot(y, finw_ref[...],
                         preferred_element_type=jnp.float32) + finb_ref[...]


def _l1_dense(msgp, degp, h1, root, bias, gamma, beta, finw, finb, ones_w):
    return pl.pallas_call(
        _l1_dense_body,
        out_shape=jax.ShapeDtypeStruct((N, C), jnp.float32),
    )(msgp, degp, h1, root, bias, gamma, beta, finw, finb, ones_w)


def kernel(x, edge_index, edge_attr, emb, W0, root0, bias0, gamma0, beta0,
           W1, root1, bias1, gamma1, beta1, finW, finb):
    f32 = jnp.float32
    src = edge_index[0]
    dst = edge_index[1]

    a0 = edge_attr[:, 0].reshape(E // C, C)
    a1 = edge_attr[:, 1].reshape(E // C, C)
    f0, f1, base = _edge_prep(a0, a1)
    f0 = f0.reshape(E)
    f1 = f1.reshape(E)
    base = base.reshape(E)

    ones_w = jnp.concatenate([jnp.ones((NS, C), f32),
                              jnp.zeros((NS, C), f32)], axis=0)
    ones_1 = jnp.ones((1, C), f32)
    xbits = jnp.sum(x.reshape(N // 16, 16) << jnp.arange(16, dtype=jnp.int32),
                    axis=1).astype(jnp.int32)
    xb2 = jnp.broadcast_to(xbits[:, None], (N // 16, 16)).reshape(N)

    w0t = W0.transpose(1, 0, 2).reshape(C, KTOT * C)
    p0 = _matmul(emb, w0t).reshape(2 * KTOT, C)

    msgp0, degp = _l0_kernel(xb2, src, dst, f0, f1, base, p0)
    degp = degp.reshape(NW, N)
    h1 = _l0_dense(msgp0, degp, x.astype(f32).reshape(1, N), emb, root0,
                   bias0.reshape(1, C), gamma0.reshape(1, C),
                   beta0.reshape(1, C), ones_w, ones_1)

    w1t = W1.transpose(1, 0, 2).reshape(C, KTOT * C)
    p1 = _p1_matmul(h1, w1t).reshape(N * KTOT, C)

    msgp1 = _l1_kernel(src, dst, f0, f1, base, p1)
    out = _l1_dense(msgp1, degp, h1, root1, bias1.reshape(1, C),
                    gamma1.reshape(1, C), beta1.reshape(1, C), finW,
                    finb.reshape(1, C), ones_w)
    return out


# speedup vs baseline: 1.7453x; 1.0061x over previous
import functools

import jax
import jax.numpy as jnp
from jax import lax
from jax.experimental import pallas as pl
from jax.experimental.pallas import tpu as pltpu
from jax.experimental.pallas import tpu_sc as plsc

N = 10000
E = 320000
C = 128
K = 5
KTOT = K * K

NC = 2
NS = 16
NW = NC * NS
EPW = E // NW
ESUB = E // NS
BB = 80
NB = ESUB // BB
HALF = N // NC
HROWS = 5120
NCH = HROWS // BB

_mesh = plsc.VectorSubcoreMesh(core_axis_name="c", subcore_axis_name="s")


def _prep_body(a0_ref, a1_ref, f0_ref, f1_ref, base_ref):
    p0 = a0_ref[...] * (K - 1.0)
    p1 = a1_ref[...] * (K - 1.0)
    lo0 = jnp.clip(jnp.floor(p0), 0.0, K - 2.0)
    lo1 = jnp.clip(jnp.floor(p1), 0.0, K - 2.0)
    f0_ref[...] = p0 - lo0
    f1_ref[...] = p1 - lo1
    base_ref[...] = lo0.astype(jnp.int32) + K * lo1.astype(jnp.int32)


def _edge_prep(a0, a1):
    return pl.pallas_call(
        _prep_body,
        out_shape=[
            jax.ShapeDtypeStruct(a0.shape, jnp.float32),
            jax.ShapeDtypeStruct(a0.shape, jnp.float32),
            jax.ShapeDtypeStruct(a0.shape, jnp.int32),
        ],
    )(a0, a1)


def _matmul_body(a_ref, b_ref, o_ref):
    o_ref[...] = jnp.dot(a_ref[...], b_ref[...],
                         preferred_element_type=jnp.float32)


def _matmul(a, b):
    return pl.pallas_call(
        _matmul_body,
        out_shape=jax.ShapeDtypeStruct((a.shape[0], b.shape[1]), jnp.float32),
    )(a, b)


def _p1_body(h_ref, w_ref, o_ref):
    o_ref[...] = jnp.dot(h_ref[...], w_ref[...],
                         preferred_element_type=jnp.float32)


def _p1_matmul(h, wt):
    rb = 1000
    return pl.pallas_call(
        _p1_body,
        grid=(N // rb,),
        in_specs=[
            pl.BlockSpec((rb, C), lambda i: (i, 0)),
            pl.BlockSpec((C, KTOT * C), lambda i: (0, 0)),
        ],
        out_specs=pl.BlockSpec((rb, KTOT * C), lambda i: (i, 0)),
        out_shape=jax.ShapeDtypeStruct((N, KTOT * C), jnp.float32),
    )(h, wt)


@functools.partial(
    pl.kernel,
    out_type=jax.ShapeDtypeStruct((NC, HROWS, C), jnp.float32),
    mesh=_mesh,
    scratch_types=[
        pltpu.VMEM_SHARED((HROWS, C), jnp.float32),
        pltpu.VMEM((2 * KTOT, C), jnp.float32),
        pltpu.VMEM((BB, C), jnp.float32),
        pltpu.VMEM((N,), jnp.int32),
        pltpu.VMEM((BB,), jnp.int32),
        pltpu.VMEM((BB,), jnp.int32),
        pltpu.VMEM((BB,), jnp.int32),
        pltpu.VMEM((BB,), jnp.float32),
        pltpu.VMEM((BB,), jnp.float32),
        pltpu.VMEM((BB, C), jnp.float32),
    ],
)
def _l0_kernel(xb2_hbm, src_hbm, dst_hbm, f0_hbm, f1_hbm, base_hbm, p0_hbm,
               msg_out,
               msg_sh, p0_v, z_v, xb_v,
               src_v, dst_v, base_v, f0_v, f1_v, rows_v):
    cid = lax.axis_index("c")
    sid = lax.axis_index("s")
    wid = cid * NS + sid

    zero16 = jnp.zeros((16,), jnp.float32)

    def zrow(i, carry):
        for c8 in range(C // 16):
            z_v[i, pl.ds(c8 * 16, 16)] = zero16
        return carry
    lax.fori_loop(0, BB, zrow, 0)

    for i in range(NCH // NS):
        ch = i * NS + sid
        pltpu.sync_copy(z_v, msg_sh.at[pl.ds(ch * BB, BB)])
    pltpu.sync_copy(p0_hbm, p0_v)
    pltpu.sync_copy(xb2_hbm, xb_v)
    plsc.subcore_barrier()

    def batch(it, carry):
        bbase = sid * ESUB + it * BB
        pltpu.sync_copy(src_hbm.at[pl.ds(bbase, BB)], src_v)
        pltpu.sync_copy(dst_hbm.at[pl.ds(bbase, BB)], dst_v)
        pltpu.sync_copy(f0_hbm.at[pl.ds(bbase, BB)], f0_v)
        pltpu.sync_copy(f1_hbm.at[pl.ds(bbase, BB)], f1_v)
        pltpu.sync_copy(base_hbm.at[pl.ds(bbase, BB)], base_v)

        def group(j, c2):
            sl = pl.ds(j * 16, 16)
            src16 = src_v[sl]
            base16 = base_v[sl]
            f016 = f0_v[sl]
            f116 = f1_v[sl]
            rel16 = dst_v[sl] - cid * HALF
            ok16 = jnp.logical_and(rel16 >= 0, rel16 < HALF)
            dst_v[sl] = jnp.where(ok16, rel16, HALF)
            for lane in range(16):
                i = j * 16 + lane
                s = src16[lane]
                xw = xb_v[pl.ds((s >> 4) * 16, 16)]
                bit = (xw[0] >> (s & 15)) & 1
                g = bit * KTOT + base16[lane]
                a0 = f016[lane]
                a1 = f116[lane]
                for c8 in range(C // 16):
                    cs = pl.ds(c8 * 16, 16)
                    va = p0_v[g, cs]
                    vb = p0_v[g + 1, cs]
                    vc = p0_v[g + K, cs]
                    vd = p0_v[g + K + 1, cs]
                    t0 = va + a0 * (vb - va)
                    t1 = vc + a0 * (vd - vc)
                    rows_v[i, cs] = t0 + a1 * (t1 - t0)
            return c2
        lax.fori_loop(0, BB // 16, group, 0)

        pltpu.sync_copy(rows_v, msg_sh.at[dst_v], add=True)
        return carry
    lax.fori_loop(0, NB, batch, 0)

    plsc.subcore_barrier()
    for i in range(NCH // NS):
        ch = i * NS + sid
        pltpu.sync_copy(msg_sh.at[pl.ds(ch * BB, BB)],
                        msg_out.at[cid, pl.ds(ch * BB, BB)])


@functools.partial(
    pl.kernel,
    out_type=jax.ShapeDtypeStruct((NW * N,), jnp.float32),
    mesh=_mesh,
    scratch_types=[
        pltpu.VMEM((N,), jnp.float32),
        pltpu.VMEM((BB,), jnp.int32),
    ],
)
def _deg_kernel(dst_hbm, deg_out, deg_v, dst_v):
    cid = lax.axis_index("c")
    sid = lax.axis_index("s")
    wid = cid * NS + sid

    zero16 = jnp.zeros((16,), jnp.float32)
    iota16 = lax.iota(jnp.int32, 16)

    def zdeg(i, carry):
        deg_v[pl.ds(i * 16, 16)] = zero16
        return carry
    lax.fori_loop(0, N // 16, zdeg, 0)

    def batch(it, carry):
        bbase = wid * EPW + it * BB
        pltpu.sync_copy(dst_hbm.at[pl.ds(bbase, BB)], dst_v)

        def group(j, c2):
            dst16 = dst_v[pl.ds(j * 16, 16)]
            for lane in range(16):
                d = dst16[lane]
                drow = (d >> 4) * 16
                dv = deg_v[pl.ds(drow, 16)]
                oh = jnp.where(iota16 == (d & 15), 1.0, 0.0)
                deg_v[pl.ds(drow, 16)] = dv + oh
            return c2
        lax.fori_loop(0, BB // 16, group, 0)
        return carry
    lax.fori_loop(0, EPW // BB, batch, 0)

    pltpu.sync_copy(deg_v, deg_out.at[pl.ds(wid * N, N)])


@functools.partial(
    pl.kernel,
    out_type=jax.ShapeDtypeStruct((NC, HROWS, C), jnp.float32),
    mesh=_mesh,
    scratch_types=[
        pltpu.VMEM_SHARED((HROWS, C), jnp.float32),
        pltpu.VMEM((BB, C), jnp.float32),
        pltpu.VMEM((BB,), jnp.int32),
        pltpu.VMEM((BB,), jnp.int32),
        pltpu.VMEM((BB,), jnp.int32),
        pltpu.VMEM((BB,), jnp.float32),
        pltpu.VMEM((BB,), jnp.float32),
        pltpu.VMEM((BB,), jnp.int32),
        pltpu.VMEM((BB,), jnp.int32),
        pltpu.VMEM((BB,), jnp.int32),
        pltpu.VMEM((BB,), jnp.int32),
        pltpu.VMEM((BB, C), jnp.float32),
        pltpu.VMEM((BB, C), jnp.float32),
        pltpu.VMEM((BB, C), jnp.float32),
        pltpu.VMEM((BB, C), jnp.float32),
        pltpu.VMEM((BB, C), jnp.float32),
        pltpu.SemaphoreType.DMA,
    ],
)
def _l1_kernel(src_hbm, dst_hbm, f0_hbm, f1_hbm, base_hbm, p1_hbm,
               msg_out,
               msg_sh, z_v, src_v, dst_v, base_v, f0_v, f1_v,
               ga_v, gb_v, gc_v, gd_v,
               ra_v, rb_v, rc_v, rd_v, rows_v, sem):
    cid = lax.axis_index("c")
    sid = lax.axis_index("s")
    wid = cid * NS + sid

    zero16 = jnp.zeros((16,), jnp.float32)

    def zrow(i, carry):
        for c8 in range(C // 16):
            z_v[i, pl.ds(c8 * 16, 16)] = zero16
        return carry
    lax.fori_loop(0, BB, zrow, 0)

    for i in range(NCH // NS):
        ch = i * NS + sid
        pltpu.sync_copy(z_v, msg_sh.at[pl.ds(ch * BB, BB)])
    plsc.subcore_barrier()

    def batch(it, carry):
        bbase = sid * ESUB + it * BB
        pltpu.sync_copy(src_hbm.at[pl.ds(bbase, BB)], src_v)
        pltpu.sync_copy(dst_hbm.at[pl.ds(bbase, BB)], dst_v)
        pltpu.sync_copy(f0_hbm.at[pl.ds(bbase, BB)], f0_v)
        pltpu.sync_copy(f1_hbm.at[pl.ds(bbase, BB)], f1_v)
        pltpu.sync_copy(base_hbm.at[pl.ds(bbase, BB)], base_v)

        for j in range(BB // 16):
            sl = pl.ds(j * 16, 16)
            g = src_v[sl] * KTOT + base_v[sl]
            ga_v[sl] = g
            gb_v[sl] = g + 1
            gc_v[sl] = g + K
            gd_v[sl] = g + K + 1
            rel16 = dst_v[sl] - cid * HALF
            ok16 = jnp.logical_and(rel16 >= 0, rel16 < HALF)
            dst_v[sl] = jnp.where(ok16, rel16, HALF)

        da = pltpu.async_copy(p1_hbm.at[ga_v], ra_v, sem)
        db = pltpu.async_copy(p1_hbm.at[gb_v], rb_v, sem)
        dc = pltpu.async_copy(p1_hbm.at[gc_v], rc_v, sem)
        dd = pltpu.async_copy(p1_hbm.at[gd_v], rd_v, sem)
        da.wait()
        db.wait()
        dc.wait()
        dd.wait()

        def group(j, c2):
            sl = pl.ds(j * 16, 16)
            f016 = f0_v[sl]
            f116 = f1_v[sl]
            for lane in range(16):
                i = j * 16 + lane
                a0 = f016[lane]
                a1 = f116[lane]
                for c8 in range(C // 16):
                    cs = pl.ds(c8 * 16, 16)
                    va = ra_v[i, cs]
                    vb = rb_v[i, cs]
                    vc = rc_v[i, cs]
                    vd = rd_v[i, cs]
                    t0 = va + a0 * (vb - va)
                    t1 = vc + a0 * (vd - vc)
                    rows_v[i, cs] = t0 + a1 * (t1 - t0)
            return c2
        lax.fori_loop(0, BB // 16, group, 0)

        pltpu.sync_copy(rows_v, msg_sh.at[dst_v], add=True)
        return carry
    lax.fori_loop(0, NB, batch, 0)

    plsc.subcore_barrier()
    for i in range(NCH // NS):
        ch = i * NS + sid
        pltpu.sync_copy(msg_sh.at[pl.ds(ch * BB, BB)],
                        msg_out.at[cid, pl.ds(ch * BB, BB)])


def _l0_dense_body(msgp_ref, degp_ref, x1_ref, emb_ref, root_ref,
                   bias_ref, gamma_ref, beta_ref, ones_w_ref, ones_1_ref,
                   h_ref):
    m = jnp.concatenate([msgp_ref[0, :HALF], msgp_ref[1, :HALF]], axis=0)
    degmat = lax.dot_general(degp_ref[...], ones_w_ref[...],
                             (((0,), (0,)), ((), ())),
                             preferred_element_type=jnp.float32)
    degmat = jnp.maximum(degmat, 1.0)
    er = jnp.dot(emb_ref[...], root_ref[...],
                 preferred_element_type=jnp.float32)
    xmat = lax.dot_general(x1_ref[...], ones_1_ref[...],
                           (((0,), (0,)), ((), ())),
                           preferred_element_type=jnp.float32)
    er0 = er[0:1, :]
    er1 = er[1:2, :]
    sel = er0 + xmat * (er1 - er0)
    msg = m / degmat + sel + bias_ref[...]
    mu = jnp.mean(msg, axis=0, keepdims=True)
    ctr = msg - mu
    var = jnp.mean(ctr * ctr, axis=0, keepdims=True)
    h = ctr * lax.rsqrt(var + 1e-5) * gamma_ref[...] + beta_ref[...]
    h_ref[...] = jnp.maximum(h, 0.0)


def _l0_dense(msgp, degp, x1, emb, root, bias, gamma, beta, ones_w, ones_1):
    return pl.pallas_call(
        _l0_dense_body,
        out_shape=jax.ShapeDtypeStruct((N, C), jnp.float32),
    )(msgp, degp, x1, emb, root, bias, gamma, beta, ones_w, ones_1)


def _l1_dense_body(msgp_ref, degp_ref, h1_ref, root_ref, bias_ref,
                   gamma_ref, beta_ref, finw_ref, finb_ref, ones_w_ref,
                   o_ref):
    m = jnp.concatenate([msgp_ref[0, :HALF], msgp_ref[1, :HALF]], axis=0)
    degmat = lax.dot_general(degp_ref[...], ones_w_ref[...],
                             (((0,), (0,)), ((), ())),
                             preferred_element_type=jnp.float32)
    degmat = jnp.maximum(degmat, 1.0)
    msg = m / degmat + jnp.dot(h1_ref[...], root_ref[...],
                               preferred_element_type=jnp.float32)
    msg = msg + bias_ref[...]
    mu = jnp.mean(msg, axis=0, keepdims=True)
    ctr = msg - mu
    var = jnp.mean(ctr * ctr, axis=0, keepdims=True)
    y = ctr * lax.rsqrt(var + 1e-5) * gamma_ref[...] + beta_ref[...]
    o_ref[...] = jnp.dot(y, finw_ref[...],
                         preferred_element_type=jnp.float32) + finb_ref[...]


def _l1_dense(msgp, degp, h1, root, bias, gamma, beta, finw, finb, ones_w):
    return pl.pallas_call(
        _l1_dense_body,
        out_shape=jax.ShapeDtypeStruct((N, C), jnp.float32),
    )(msgp, degp, h1, root, bias, gamma, beta, finw, finb, ones_w)


def kernel(x, edge_index, edge_attr, emb, W0, root0, bias0, gamma0, beta0,
           W1, root1, bias1, gamma1, beta1, finW, finb):
    f32 = jnp.float32
    src = edge_index[0]
    dst = edge_index[1]

    a0 = edge_attr[:, 0].reshape(E // C, C)
    a1 = edge_attr[:, 1].reshape(E // C, C)
    f0, f1, base = _edge_prep(a0, a1)
    f0 = f0.reshape(E)
    f1 = f1.reshape(E)
    base = base.reshape(E)

    ones_w = jnp.ones((NW, C), f32)
    ones_1 = jnp.ones((1, C), f32)
    xbits = jnp.sum(x.reshape(N // 16, 16) << jnp.arange(16, dtype=jnp.int32),
                    axis=1).astype(jnp.int32)
    xb2 = jnp.broadcast_to(xbits[:, None], (N // 16, 16)).reshape(N)

    w0t = W0.transpose(1, 0, 2).reshape(C, KTOT * C)
    p0 = _matmul(emb, w0t).reshape(2 * KTOT, C)

    msgp0 = _l0_kernel(xb2, src, dst, f0, f1, base, p0)
    degp = _deg_kernel(dst).reshape(NW, N)
    h1 = _l0_dense(msgp0, degp, x.astype(f32).reshape(1, N), emb, root0,
                   bias0.reshape(1, C), gamma0.reshape(1, C),
                   beta0.reshape(1, C), ones_w, ones_1)

    w1t = W1.transpose(1, 0, 2).reshape(C, KTOT * C)
    p1 = _p1_matmul(h1, w1t).reshape(N * KTOT, C)

    msgp1 = _l1_kernel(src, dst, f0, f1, base, p1)
    out = _l1_dense(msgp1, degp, h1, root1, bias1.reshape(1, C),
                    gamma1.reshape(1, C), beta1.reshape(1, C), finW,
                    finb.reshape(1, C), ones_w)
    return out


# speedup vs baseline: 2.2616x; 1.2958x over previous
import functools

import jax
import jax.numpy as jnp
from jax import lax
from jax.experimental import pallas as pl
from jax.experimental.pallas import tpu as pltpu
from jax.experimental.pallas import tpu_sc as plsc

N = 10000
E = 320000
C = 128
K = 5
KTOT = K * K

NC = 2
NS = 16
NW = NC * NS
EPW = E // NW
ESUB = E // NS
BB = 80
NB = ESUB // BB
HALF = N // NC
HROWS = 5120
NCH = HROWS // BB

_mesh = plsc.VectorSubcoreMesh(core_axis_name="c", subcore_axis_name="s")


def _prep_body(a0_ref, a1_ref, f0_ref, f1_ref, base_ref):
    p0 = a0_ref[...] * (K - 1.0)
    p1 = a1_ref[...] * (K - 1.0)
    lo0 = jnp.clip(jnp.floor(p0), 0.0, K - 2.0)
    lo1 = jnp.clip(jnp.floor(p1), 0.0, K - 2.0)
    f0_ref[...] = p0 - lo0
    f1_ref[...] = p1 - lo1
    base_ref[...] = lo0.astype(jnp.int32) + K * lo1.astype(jnp.int32)


def _edge_prep(a0, a1):
    return pl.pallas_call(
        _prep_body,
        out_shape=[
            jax.ShapeDtypeStruct(a0.shape, jnp.float32),
            jax.ShapeDtypeStruct(a0.shape, jnp.float32),
            jax.ShapeDtypeStruct(a0.shape, jnp.int32),
        ],
    )(a0, a1)


def _matmul_body(a_ref, b_ref, o_ref):
    o_ref[...] = jnp.dot(a_ref[...], b_ref[...],
                         preferred_element_type=jnp.float32)


def _matmul(a, b):
    return pl.pallas_call(
        _matmul_body,
        out_shape=jax.ShapeDtypeStruct((a.shape[0], b.shape[1]), jnp.float32),
    )(a, b)


def _p1_body(h_ref, w_ref, o_ref):
    o_ref[...] = jnp.dot(h_ref[...], w_ref[...],
                         preferred_element_type=jnp.float32)


def _p1_matmul(h, wt):
    rb = 1000
    return pl.pallas_call(
        _p1_body,
        grid=(N // rb,),
        in_specs=[
            pl.BlockSpec((rb, C), lambda i: (i, 0)),
            pl.BlockSpec((C, KTOT * C), lambda i: (0, 0)),
        ],
        out_specs=pl.BlockSpec((rb, KTOT * C), lambda i: (i, 0)),
        out_shape=jax.ShapeDtypeStruct((N, KTOT * C), jnp.float32),
    )(h, wt)


@functools.partial(
    pl.kernel,
    out_type=jax.ShapeDtypeStruct((NC, HROWS, C), jnp.float32),
    mesh=_mesh,
    scratch_types=[
        pltpu.VMEM_SHARED((HROWS, C), jnp.float32),
        pltpu.VMEM((2 * KTOT, C), jnp.float32),
        pltpu.VMEM((BB, C), jnp.float32),
        pltpu.VMEM((BB,), jnp.int32),
        pltpu.VMEM((BB,), jnp.int32),
        pltpu.VMEM((BB,), jnp.float32),
        pltpu.VMEM((BB,), jnp.float32),
        pltpu.VMEM((BB, C), jnp.float32),
    ],
)
def _l0_kernel(dst_hbm, f0_hbm, f1_hbm, g_hbm, p0_hbm,
               msg_out,
               msg_sh, p0_v, z_v,
               dst_v, base_v, f0_v, f1_v, rows_v):
    cid = lax.axis_index("c")
    sid = lax.axis_index("s")
    wid = cid * NS + sid

    zero16 = jnp.zeros((16,), jnp.float32)

    def zrow(i, carry):
        for c8 in range(C // 16):
            z_v[i, pl.ds(c8 * 16, 16)] = zero16
        return carry
    lax.fori_loop(0, BB, zrow, 0)

    for i in range(NCH // NS):
        ch = i * NS + sid
        pltpu.sync_copy(z_v, msg_sh.at[pl.ds(ch * BB, BB)])
    pltpu.sync_copy(p0_hbm, p0_v)
    plsc.subcore_barrier()

    def batch(it, carry):
        bbase = sid * ESUB + it * BB
        pltpu.sync_copy(dst_hbm.at[pl.ds(bbase, BB)], dst_v)
        pltpu.sync_copy(f0_hbm.at[pl.ds(bbase, BB)], f0_v)
        pltpu.sync_copy(f1_hbm.at[pl.ds(bbase, BB)], f1_v)
        pltpu.sync_copy(g_hbm.at[pl.ds(bbase, BB)], base_v)

        def group(j, c2):
            sl = pl.ds(j * 16, 16)
            base16 = base_v[sl]
            f016 = f0_v[sl]
            f116 = f1_v[sl]
            rel16 = dst_v[sl] - cid * HALF
            ok16 = jnp.logical_and(rel16 >= 0, rel16 < HALF)
            dst_v[sl] = jnp.where(ok16, rel16, HALF)
            for lane in range(16):
                i = j * 16 + lane
                g = base16[lane]
                a0 = f016[lane]
                a1 = f116[lane]
                for c8 in range(C // 16):
                    cs = pl.ds(c8 * 16, 16)
                    va = p0_v[g, cs]
                    vb = p0_v[g + 1, cs]
                    vc = p0_v[g + K, cs]
                    vd = p0_v[g + K + 1, cs]
                    t0 = va + a0 * (vb - va)
                    t1 = vc + a0 * (vd - vc)
                    rows_v[i, cs] = t0 + a1 * (t1 - t0)
            return c2
        lax.fori_loop(0, BB // 16, group, 0)

        pltpu.sync_copy(rows_v, msg_sh.at[dst_v], add=True)
        return carry
    lax.fori_loop(0, NB, batch, 0)

    plsc.subcore_barrier()
    for i in range(NCH // NS):
        ch = i * NS + sid
        pltpu.sync_copy(msg_sh.at[pl.ds(ch * BB, BB)],
                        msg_out.at[cid, pl.ds(ch * BB, BB)])


@functools.partial(
    pl.kernel,
    out_type=[
        jax.ShapeDtypeStruct((NW * N,), jnp.float32),
        jax.ShapeDtypeStruct((E,), jnp.int32),
    ],
    mesh=_mesh,
    scratch_types=[
        pltpu.VMEM((N,), jnp.float32),
        pltpu.VMEM((N,), jnp.int32),
        pltpu.VMEM((BB,), jnp.int32),
        pltpu.VMEM((BB,), jnp.int32),
        pltpu.VMEM((BB,), jnp.int32),
    ],
)
def _deg_kernel(dst_hbm, src_hbm, base_hbm, xb2_hbm, deg_out, g_out,
                deg_v, xb_v, dst_v, src_v, base_v):
    cid = lax.axis_index("c")
    sid = lax.axis_index("s")
    wid = cid * NS + sid

    zero16 = jnp.zeros((16,), jnp.float32)
    iota16 = lax.iota(jnp.int32, 16)

    def zdeg(i, carry):
        deg_v[pl.ds(i * 16, 16)] = zero16
        return carry
    lax.fori_loop(0, N // 16, zdeg, 0)
    pltpu.sync_copy(xb2_hbm, xb_v)

    def batch(it, carry):
        bbase = wid * EPW + it * BB
        pltpu.sync_copy(dst_hbm.at[pl.ds(bbase, BB)], dst_v)
        pltpu.sync_copy(src_hbm.at[pl.ds(bbase, BB)], src_v)
        pltpu.sync_copy(base_hbm.at[pl.ds(bbase, BB)], base_v)

        def group(j, c2):
            sl = pl.ds(j * 16, 16)
            dst16 = dst_v[sl]
            src16 = src_v[sl]
            gacc = base_v[sl]
            for lane in range(16):
                d = dst16[lane]
                drow = (d >> 4) * 16
                dv = deg_v[pl.ds(drow, 16)]
                oh = jnp.where(iota16 == (d & 15), 1.0, 0.0)
                deg_v[pl.ds(drow, 16)] = dv + oh
                s = src16[lane]
                xw = xb_v[pl.ds((s >> 4) * 16, 16)]
                bit = (xw[0] >> (s & 15)) & 1
                gacc = gacc + jnp.where(iota16 == lane, bit * KTOT, 0)
            base_v[sl] = gacc
            return c2
        lax.fori_loop(0, BB // 16, group, 0)

        pltpu.sync_copy(base_v, g_out.at[pl.ds(bbase, BB)])
        return carry
    lax.fori_loop(0, EPW // BB, batch, 0)

    pltpu.sync_copy(deg_v, deg_out.at[pl.ds(wid * N, N)])


@functools.partial(
    pl.kernel,
    out_type=jax.ShapeDtypeStruct((NC, HROWS, C), jnp.float32),
    mesh=_mesh,
    scratch_types=[
        pltpu.VMEM_SHARED((HROWS, C), jnp.float32),
        pltpu.VMEM((BB, C), jnp.float32),
        pltpu.VMEM((BB,), jnp.int32),
        pltpu.VMEM((BB,), jnp.int32),
        pltpu.VMEM((BB,), jnp.int32),
        pltpu.VMEM((BB,), jnp.float32),
        pltpu.VMEM((BB,), jnp.float32),
        pltpu.VMEM((BB,), jnp.int32),
        pltpu.VMEM((BB,), jnp.int32),
        pltpu.VMEM((BB,), jnp.int32),
        pltpu.VMEM((BB,), jnp.int32),
        pltpu.VMEM((BB, C), jnp.float32),
        pltpu.VMEM((BB, C), jnp.float32),
        pltpu.VMEM((BB, C), jnp.float32),
        pltpu.VMEM((BB, C), jnp.float32),
        pltpu.VMEM((BB, C), jnp.float32),
        pltpu.SemaphoreType.DMA,
    ],
)
def _l1_kernel(src_hbm, dst_hbm, f0_hbm, f1_hbm, base_hbm, p1_hbm,
               msg_out,
               msg_sh, z_v, src_v, dst_v, base_v, f0_v, f1_v,
               ga_v, gb_v, gc_v, gd_v,
               ra_v, rb_v, rc_v, rd_v, rows_v, sem):
    cid = lax.axis_index("c")
    sid = lax.axis_index("s")
    wid = cid * NS + sid

    zero16 = jnp.zeros((16,), jnp.float32)

    def zrow(i, carry):
        for c8 in range(C // 16):
            z_v[i, pl.ds(c8 * 16, 16)] = zero16
        return carry
    lax.fori_loop(0, BB, zrow, 0)

    for i in range(NCH // NS):
        ch = i * NS + sid
        pltpu.sync_copy(z_v, msg_sh.at[pl.ds(ch * BB, BB)])
    plsc.subcore_barrier()

    def batch(it, carry):
        bbase = sid * ESUB + it * BB
        pltpu.sync_copy(src_hbm.at[pl.ds(bbase, BB)], src_v)
        pltpu.sync_copy(dst_hbm.at[pl.ds(bbase, BB)], dst_v)
        pltpu.sync_copy(f0_hbm.at[pl.ds(bbase, BB)], f0_v)
        pltpu.sync_copy(f1_hbm.at[pl.ds(bbase, BB)], f1_v)
        pltpu.sync_copy(base_hbm.at[pl.ds(bbase, BB)], base_v)

        for j in range(BB // 16):
            sl = pl.ds(j * 16, 16)
            g = src_v[sl] * KTOT + base_v[sl]
            ga_v[sl] = g
            gb_v[sl] = g + 1
            gc_v[sl] = g + K
            gd_v[sl] = g + K + 1
            rel16 = dst_v[sl] - cid * HALF
            ok16 = jnp.logical_and(rel16 >= 0, rel16 < HALF)
            dst_v[sl] = jnp.where(ok16, rel16, HALF)

        da = pltpu.async_copy(p1_hbm.at[ga_v], ra_v, sem)
        db = pltpu.async_copy(p1_hbm.at[gb_v], rb_v, sem)
        dc = pltpu.async_copy(p1_hbm.at[gc_v], rc_v, sem)
        dd = pltpu.async_copy(p1_hbm.at[gd_v], rd_v, sem)
        da.wait()
        db.wait()
        dc.wait()
        dd.wait()

        def group(j, c2):
            sl = pl.ds(j * 16, 16)
            f016 = f0_v[sl]
            f116 = f1_v[sl]
            for lane in range(16):
                i = j * 16 + lane
                a0 = f016[lane]
                a1 = f116[lane]
                for c8 in range(C // 16):
                    cs = pl.ds(c8 * 16, 16)
                    va = ra_v[i, cs]
                    vb = rb_v[i, cs]
                    vc = rc_v[i, cs]
                    vd = rd_v[i, cs]
                    t0 = va + a0 * (vb - va)
                    t1 = vc + a0 * (vd - vc)
                    rows_v[i, cs] = t0 + a1 * (t1 - t0)
            return c2
        lax.fori_loop(0, BB // 16, group, 0)

        pltpu.sync_copy(rows_v, msg_sh.at[dst_v], add=True)
        return carry
    lax.fori_loop(0, NB, batch, 0)

    plsc.subcore_barrier()
    for i in range(NCH // NS):
        ch = i * NS + sid
        pltpu.sync_copy(msg_sh.at[pl.ds(ch * BB, BB)],
                        msg_out.at[cid, pl.ds(ch * BB, BB)])


def _l0_dense_body(msgp_ref, degp_ref, x1_ref, emb_ref, root_ref,
                   bias_ref, gamma_ref, beta_ref, ones_w_ref, ones_1_ref,
                   h_ref):
    m = jnp.concatenate([msgp_ref[0, :HALF], msgp_ref[1, :HALF]], axis=0)
    degmat = lax.dot_general(degp_ref[...], ones_w_ref[...],
                             (((0,), (0,)), ((), ())),
                             preferred_element_type=jnp.float32)
    degmat = jnp.maximum(degmat, 1.0)
    er = jnp.dot(emb_ref[...], root_ref[...],
                 preferred_element_type=jnp.float32)
    xmat = lax.dot_general(x1_ref[...], ones_1_ref[...],
                           (((0,), (0,)), ((), ())),
                           preferred_element_type=jnp.float32)
    er0 = er[0:1, :]
    er1 = er[1:2, :]
    sel = er0 + xmat * (er1 - er0)
    msg = m / degmat + sel + bias_ref[...]
    mu = jnp.mean(msg, axis=0, keepdims=True)
    ctr = msg - mu
    var = jnp.mean(ctr * ctr, axis=0, keepdims=True)
    h = ctr * lax.rsqrt(var + 1e-5) * gamma_ref[...] + beta_ref[...]
    h_ref[...] = jnp.maximum(h, 0.0)


def _l0_dense(msgp, degp, x1, emb, root, bias, gamma, beta, ones_w, ones_1):
    return pl.pallas_call(
        _l0_dense_body,
        out_shape=jax.ShapeDtypeStruct((N, C), jnp.float32),
    )(msgp, degp, x1, emb, root, bias, gamma, beta, ones_w, ones_1)


def _l1_dense_body(msgp_ref, degp_ref, h1_ref, root_ref, bias_ref,
                   gamma_ref, beta_ref, finw_ref, finb_ref, ones_w_ref,
                   o_ref):
    m = jnp.concatenate([msgp_ref[0, :HALF], msgp_ref[1, :HALF]], axis=0)
    degmat = lax.dot_general(degp_ref[...], ones_w_ref[...],
                             (((0,), (0,)), ((), ())),
                             preferred_element_type=jnp.float32)
    degmat = jnp.maximum(degmat, 1.0)
    msg = m / degmat + jnp.dot(h1_ref[...], root_ref[...],
                               preferred_element_type=jnp.float32)
    msg = msg + bias_ref[...]
    mu = jnp.mean(msg, axis=0, keepdims=True)
    ctr = msg - mu
    var = jnp.mean(ctr * ctr, axis=0, keepdims=True)
    y = ctr * lax.rsqrt(var + 1e-5) * gamma_ref[...] + beta_ref[...]
    o_ref[...] = jnp.dot(y, finw_ref[...],
                         preferred_element_type=jnp.float32) + finb_ref[...]


def _l1_dense(msgp, degp, h1, root, bias, gamma, beta, finw, finb, ones_w):
    return pl.pallas_call(
        _l1_dense_body,
        out_shape=jax.ShapeDtypeStruct((N, C), jnp.float32),
    )(msgp, degp, h1, root, bias, gamma, beta, finw, finb, ones_w)


def kernel(x, edge_index, edge_attr, emb, W0, root0, bias0, gamma0, beta0,
           W1, root1, bias1, gamma1, beta1, finW, finb):
    f32 = jnp.float32
    src = edge_index[0]
    dst = edge_index[1]

    a0 = edge_attr[:, 0].reshape(E // C, C)
    a1 = edge_attr[:, 1].reshape(E // C, C)
    f0, f1, base = _edge_prep(a0, a1)
    f0 = f0.reshape(E)
    f1 = f1.reshape(E)
    base = base.reshape(E)

    ones_w = jnp.ones((NW, C), f32)
    ones_1 = jnp.ones((1, C), f32)
    xbits = jnp.sum(x.reshape(N // 16, 16) << jnp.arange(16, dtype=jnp.int32),
                    axis=1).astype(jnp.int32)
    xb2 = jnp.broadcast_to(xbits[:, None], (N // 16, 16)).reshape(N)

    w0t = W0.transpose(1, 0, 2).reshape(C, KTOT * C)
    p0 = _matmul(emb, w0t).reshape(2 * KTOT, C)

    degp, gfull = _deg_kernel(dst, src, base, xb2)
    degp = degp.reshape(NW, N)
    msgp0 = _l0_kernel(dst, f0, f1, gfull, p0)
    h1 = _l0_dense(msgp0, degp, x.astype(f32).reshape(1, N), emb, root0,
                   bias0.reshape(1, C), gamma0.reshape(1, C),
                   beta0.reshape(1, C), ones_w, ones_1)

    w1t = W1.transpose(1, 0, 2).reshape(C, KTOT * C)
    p1 = _p1_matmul(h1, w1t).reshape(N * KTOT, C)

    msgp1 = _l1_kernel(src, dst, f0, f1, base, p1)
    out = _l1_dense(msgp1, degp, h1, root1, bias1.reshape(1, C),
                    gamma1.reshape(1, C), beta1.reshape(1, C), finW,
                    finb.reshape(1, C), ones_w)
    return out


# speedup vs baseline: 2.2696x; 1.0036x over previous
import functools

import jax
import jax.numpy as jnp
from jax import lax
from jax.experimental import pallas as pl
from jax.experimental.pallas import tpu as pltpu
from jax.experimental.pallas import tpu_sc as plsc

N = 10000
E = 320000
C = 128
K = 5
KTOT = K * K

NC = 2
NS = 16
NW = NC * NS
EPW = E // NW
ESUB = E // NS
BB = 80
NB = ESUB // BB
HALF = N // NC
HROWS = 5120
NCH = HROWS // BB

_mesh = plsc.VectorSubcoreMesh(core_axis_name="c", subcore_axis_name="s")


def _prep_body(a0_ref, a1_ref, f0_ref, f1_ref, base_ref):
    p0 = a0_ref[...] * (K - 1.0)
    p1 = a1_ref[...] * (K - 1.0)
    lo0 = jnp.clip(jnp.floor(p0), 0.0, K - 2.0)
    lo1 = jnp.clip(jnp.floor(p1), 0.0, K - 2.0)
    f0_ref[...] = p0 - lo0
    f1_ref[...] = p1 - lo1
    base_ref[...] = lo0.astype(jnp.int32) + K * lo1.astype(jnp.int32)


def _edge_prep(a0, a1):
    return pl.pallas_call(
        _prep_body,
        out_shape=[
            jax.ShapeDtypeStruct(a0.shape, jnp.float32),
            jax.ShapeDtypeStruct(a0.shape, jnp.float32),
            jax.ShapeDtypeStruct(a0.shape, jnp.int32),
        ],
    )(a0, a1)


def _matmul_body(a_ref, b_ref, o_ref):
    o_ref[...] = jnp.dot(a_ref[...], b_ref[...],
                         preferred_element_type=jnp.float32)


def _matmul(a, b):
    return pl.pallas_call(
        _matmul_body,
        out_shape=jax.ShapeDtypeStruct((a.shape[0], b.shape[1]), jnp.float32),
    )(a, b)


def _p1_body(h_ref, w_ref, o_ref):
    o_ref[...] = jnp.dot(h_ref[...], w_ref[...],
                         preferred_element_type=jnp.float32)


def _p1_matmul(h, wt):
    rb = 1000
    return pl.pallas_call(
        _p1_body,
        grid=(N // rb,),
        in_specs=[
            pl.BlockSpec((rb, C), lambda i: (i, 0)),
            pl.BlockSpec((C, KTOT * C), lambda i: (0, 0)),
        ],
        out_specs=pl.BlockSpec((rb, KTOT * C), lambda i: (i, 0)),
        out_shape=jax.ShapeDtypeStruct((N, KTOT * C), jnp.float32),
    )(h, wt)


@functools.partial(
    pl.kernel,
    out_type=jax.ShapeDtypeStruct((NC, HROWS, C), jnp.float32),
    mesh=_mesh,
    scratch_types=[
        pltpu.VMEM_SHARED((HROWS, C), jnp.float32),
        pltpu.VMEM((2 * KTOT, C), jnp.float32),
        pltpu.VMEM((BB, C), jnp.float32),
        pltpu.VMEM((BB,), jnp.int32),
        pltpu.VMEM((BB,), jnp.int32),
        pltpu.VMEM((BB,), jnp.float32),
        pltpu.VMEM((BB,), jnp.float32),
        pltpu.VMEM((BB, C), jnp.float32),
    ],
)
def _l0_kernel(dst_hbm, f0_hbm, f1_hbm, g_hbm, p0_hbm,
               msg_out,
               msg_sh, p0_v, z_v,
               dst_v, base_v, f0_v, f1_v, rows_v):
    cid = lax.axis_index("c")
    sid = lax.axis_index("s")
    wid = cid * NS + sid

    zero16 = jnp.zeros((16,), jnp.float32)

    def zrow(i, carry):
        for c8 in range(C // 16):
            z_v[i, pl.ds(c8 * 16, 16)] = zero16
        return carry
    lax.fori_loop(0, BB, zrow, 0)

    for i in range(NCH // NS):
        ch = i * NS + sid
        pltpu.sync_copy(z_v, msg_sh.at[pl.ds(ch * BB, BB)])
    pltpu.sync_copy(p0_hbm, p0_v)
    plsc.subcore_barrier()

    def batch(it, carry):
        bbase = sid * ESUB + it * BB
        pltpu.sync_copy(dst_hbm.at[pl.ds(bbase, BB)], dst_v)
        pltpu.sync_copy(f0_hbm.at[pl.ds(bbase, BB)], f0_v)
        pltpu.sync_copy(f1_hbm.at[pl.ds(bbase, BB)], f1_v)
        pltpu.sync_copy(g_hbm.at[pl.ds(bbase, BB)], base_v)

        def group(j, c2):
            sl = pl.ds(j * 16, 16)
            base16 = base_v[sl]
            f016 = f0_v[sl]
            f116 = f1_v[sl]
            rel16 = dst_v[sl] - cid * HALF
            ok16 = jnp.logical_and(rel16 >= 0, rel16 < HALF)
            dst_v[sl] = jnp.where(ok16, rel16, HALF)
            for lane in range(16):
                i = j * 16 + lane
                g = base16[lane]
                a0 = f016[lane]
                a1 = f116[lane]
                for c8 in range(C // 16):
                    cs = pl.ds(c8 * 16, 16)
                    va = p0_v[g, cs]
                    vb = p0_v[g + 1, cs]
                    vc = p0_v[g + K, cs]
                    vd = p0_v[g + K + 1, cs]
                    t0 = va + a0 * (vb - va)
                    t1 = vc + a0 * (vd - vc)
                    rows_v[i, cs] = t0 + a1 * (t1 - t0)
            return c2
        lax.fori_loop(0, BB // 16, group, 0)

        pltpu.sync_copy(rows_v,
                        msg_sh.at[plsc.Indices(dst_v, ignored_value=HALF)],
                        add=True)
        return carry
    lax.fori_loop(0, NB, batch, 0)

    plsc.subcore_barrier()
    for i in range(NCH // NS):
        ch = i * NS + sid
        pltpu.sync_copy(msg_sh.at[pl.ds(ch * BB, BB)],
                        msg_out.at[cid, pl.ds(ch * BB, BB)])


@functools.partial(
    pl.kernel,
    out_type=[
        jax.ShapeDtypeStruct((NW * N,), jnp.float32),
        jax.ShapeDtypeStruct((E,), jnp.int32),
    ],
    mesh=_mesh,
    scratch_types=[
        pltpu.VMEM((N,), jnp.float32),
        pltpu.VMEM((N,), jnp.int32),
        pltpu.VMEM((BB,), jnp.int32),
        pltpu.VMEM((BB,), jnp.int32),
        pltpu.VMEM((BB,), jnp.int32),
    ],
)
def _deg_kernel(dst_hbm, src_hbm, base_hbm, xb2_hbm, deg_out, g_out,
                deg_v, xb_v, dst_v, src_v, base_v):
    cid = lax.axis_index("c")
    sid = lax.axis_index("s")
    wid = cid * NS + sid

    zero16 = jnp.zeros((16,), jnp.float32)
    iota16 = lax.iota(jnp.int32, 16)

    def zdeg(i, carry):
        deg_v[pl.ds(i * 16, 16)] = zero16
        return carry
    lax.fori_loop(0, N // 16, zdeg, 0)
    pltpu.sync_copy(xb2_hbm, xb_v)

    def batch(it, carry):
        bbase = wid * EPW + it * BB
        pltpu.sync_copy(dst_hbm.at[pl.ds(bbase, BB)], dst_v)
        pltpu.sync_copy(src_hbm.at[pl.ds(bbase, BB)], src_v)
        pltpu.sync_copy(base_hbm.at[pl.ds(bbase, BB)], base_v)

        def group(j, c2):
            sl = pl.ds(j * 16, 16)
            dst16 = dst_v[sl]
            src16 = src_v[sl]
            gacc = base_v[sl]
            for lane in range(16):
                d = dst16[lane]
                drow = (d >> 4) * 16
                dv = deg_v[pl.ds(drow, 16)]
                oh = jnp.where(iota16 == (d & 15), 1.0, 0.0)
                deg_v[pl.ds(drow, 16)] = dv + oh
                s = src16[lane]
                xw = xb_v[pl.ds((s >> 4) * 16, 16)]
                bit = (xw[0] >> (s & 15)) & 1
                gacc = gacc + jnp.where(iota16 == lane, bit * KTOT, 0)
            base_v[sl] = gacc
            return c2
        lax.fori_loop(0, BB // 16, group, 0)

        pltpu.sync_copy(base_v, g_out.at[pl.ds(bbase, BB)])
        return carry
    lax.fori_loop(0, EPW // BB, batch, 0)

    pltpu.sync_copy(deg_v, deg_out.at[pl.ds(wid * N, N)])


@functools.partial(
    pl.kernel,
    out_type=jax.ShapeDtypeStruct((NC, HROWS, C), jnp.float32),
    mesh=_mesh,
    scratch_types=[
        pltpu.VMEM_SHARED((HROWS, C), jnp.float32),
        pltpu.VMEM((BB, C), jnp.float32),
        pltpu.VMEM((BB,), jnp.int32),
        pltpu.VMEM((BB,), jnp.int32),
        pltpu.VMEM((BB,), jnp.int32),
        pltpu.VMEM((BB,), jnp.float32),
        pltpu.VMEM((BB,), jnp.float32),
        pltpu.VMEM((BB,), jnp.int32),
        pltpu.VMEM((BB,), jnp.int32),
        pltpu.VMEM((BB,), jnp.int32),
        pltpu.VMEM((BB,), jnp.int32),
        pltpu.VMEM((BB, C), jnp.float32),
        pltpu.VMEM((BB, C), jnp.float32),
        pltpu.VMEM((BB, C), jnp.float32),
        pltpu.VMEM((BB, C), jnp.float32),
        pltpu.VMEM((BB, C), jnp.float32),
        pltpu.SemaphoreType.DMA,
    ],
)
def _l1_kernel(src_hbm, dst_hbm, f0_hbm, f1_hbm, base_hbm, p1_hbm,
               msg_out,
               msg_sh, z_v, src_v, dst_v, base_v, f0_v, f1_v,
               ga_v, gb_v, gc_v, gd_v,
               ra_v, rb_v, rc_v, rd_v, rows_v, sem):
    cid = lax.axis_index("c")
    sid = lax.axis_index("s")
    wid = cid * NS + sid

    zero16 = jnp.zeros((16,), jnp.float32)

    def zrow(i, carry):
        for c8 in range(C // 16):
            z_v[i, pl.ds(c8 * 16, 16)] = zero16
        return carry
    lax.fori_loop(0, BB, zrow, 0)

    for i in range(NCH // NS):
        ch = i * NS + sid
        pltpu.sync_copy(z_v, msg_sh.at[pl.ds(ch * BB, BB)])
    plsc.subcore_barrier()

    def batch(it, carry):
        bbase = sid * ESUB + it * BB
        pltpu.sync_copy(src_hbm.at[pl.ds(bbase, BB)], src_v)
        pltpu.sync_copy(dst_hbm.at[pl.ds(bbase, BB)], dst_v)
        pltpu.sync_copy(f0_hbm.at[pl.ds(bbase, BB)], f0_v)
        pltpu.sync_copy(f1_hbm.at[pl.ds(bbase, BB)], f1_v)
        pltpu.sync_copy(base_hbm.at[pl.ds(bbase, BB)], base_v)

        for j in range(BB // 16):
            sl = pl.ds(j * 16, 16)
            rel16 = dst_v[sl] - cid * HALF
            ok16 = jnp.logical_and(rel16 >= 0, rel16 < HALF)
            dst_v[sl] = jnp.where(ok16, rel16, HALF)
            g = jnp.where(ok16, src_v[sl] * KTOT + base_v[sl], N * KTOT)
            ga_v[sl] = g
            gb_v[sl] = jnp.where(ok16, g + 1, N * KTOT)
            gc_v[sl] = jnp.where(ok16, g + K, N * KTOT)
            gd_v[sl] = jnp.where(ok16, g + K + 1, N * KTOT)

        da = pltpu.async_copy(
            p1_hbm.at[plsc.Indices(ga_v, ignored_value=N * KTOT)], ra_v, sem)
        db = pltpu.async_copy(
            p1_hbm.at[plsc.Indices(gb_v, ignored_value=N * KTOT)], rb_v, sem)
        dc = pltpu.async_copy(
            p1_hbm.at[plsc.Indices(gc_v, ignored_value=N * KTOT)], rc_v, sem)
        dd = pltpu.async_copy(
            p1_hbm.at[plsc.Indices(gd_v, ignored_value=N * KTOT)], rd_v, sem)
        da.wait()
        db.wait()
        dc.wait()
        dd.wait()

        def group(j, c2):
            sl = pl.ds(j * 16, 16)
            f016 = f0_v[sl]
            f116 = f1_v[sl]
            for lane in range(16):
                i = j * 16 + lane
                a0 = f016[lane]
                a1 = f116[lane]
                for c8 in range(C // 16):
                    cs = pl.ds(c8 * 16, 16)
                    va = ra_v[i, cs]
                    vb = rb_v[i, cs]
                    vc = rc_v[i, cs]
                    vd = rd_v[i, cs]
                    t0 = va + a0 * (vb - va)
                    t1 = vc + a0 * (vd - vc)
                    rows_v[i, cs] = t0 + a1 * (t1 - t0)
            return c2
        lax.fori_loop(0, BB // 16, group, 0)

        pltpu.sync_copy(rows_v,
                        msg_sh.at[plsc.Indices(dst_v, ignored_value=HALF)],
                        add=True)
        return carry
    lax.fori_loop(0, NB, batch, 0)

    plsc.subcore_barrier()
    for i in range(NCH // NS):
        ch = i * NS + sid
        pltpu.sync_copy(msg_sh.at[pl.ds(ch * BB, BB)],
                        msg_out.at[cid, pl.ds(ch * BB, BB)])


def _l0_dense_body(msgp_ref, degp_ref, x1_ref, emb_ref, root_ref,
                   bias_ref, gamma_ref, beta_ref, ones_w_ref, ones_1_ref,
                   h_ref):
    m = jnp.concatenate([msgp_ref[0, :HALF], msgp_ref[1, :HALF]], axis=0)
    degmat = lax.dot_general(degp_ref[...], ones_w_ref[...],
                             (((0,), (0,)), ((), ())),
                             preferred_element_type=jnp.float32)
    degmat = jnp.maximum(degmat, 1.0)
    er = jnp.dot(emb_ref[...], root_ref[...],
                 preferred_element_type=jnp.float32)
    xmat = lax.dot_general(x1_ref[...], ones_1_ref[...],
                           (((0,), (0,)), ((), ())),
                           preferred_element_type=jnp.float32)
    er0 = er[0:1, :]
    er1 = er[1:2, :]
    sel = er0 + xmat * (er1 - er0)
    msg = m / degmat + sel + bias_ref[...]
    mu = jnp.mean(msg, axis=0, keepdims=True)
    ctr = msg - mu
    var = jnp.mean(ctr * ctr, axis=0, keepdims=True)
    h = ctr * lax.rsqrt(var + 1e-5) * gamma_ref[...] + beta_ref[...]
    h_ref[...] = jnp.maximum(h, 0.0)


def _l0_dense(msgp, degp, x1, emb, root, bias, gamma, beta, ones_w, ones_1):
    return pl.pallas_call(
        _l0_dense_body,
        out_shape=jax.ShapeDtypeStruct((N, C), jnp.float32),
    )(msgp, degp, x1, emb, root, bias, gamma, beta, ones_w, ones_1)


def _l1_dense_body(msgp_ref, degp_ref, h1_ref, root_ref, bias_ref,
                   gamma_ref, beta_ref, finw_ref, finb_ref, ones_w_ref,
                   o_ref):
    m = jnp.concatenate([msgp_ref[0, :HALF], msgp_ref[1, :HALF]], axis=0)
    degmat = lax.dot_general(degp_ref[...], ones_w_ref[...],
                             (((0,), (0,)), ((), ())),
                             preferred_element_type=jnp.float32)
    degmat = jnp.maximum(degmat, 1.0)
    msg = m / degmat + jnp.dot(h1_ref[...], root_ref[...],
                               preferred_element_type=jnp.float32)
    msg = msg + bias_ref[...]
    mu = jnp.mean(msg, axis=0, keepdims=True)
    ctr = msg - mu
    var = jnp.mean(ctr * ctr, axis=0, keepdims=True)
    y = ctr * lax.rsqrt(var + 1e-5) * gamma_ref[...] + beta_ref[...]
    o_ref[...] = jnp.dot(y, finw_ref[...],
                         preferred_element_type=jnp.float32) + finb_ref[...]


def _l1_dense(msgp, degp, h1, root, bias, gamma, beta, finw, finb, ones_w):
    return pl.pallas_call(
        _l1_dense_body,
        out_shape=jax.ShapeDtypeStruct((N, C), jnp.float32),
    )(msgp, degp, h1, root, bias, gamma, beta, finw, finb, ones_w)


def kernel(x, edge_index, edge_attr, emb, W0, root0, bias0, gamma0, beta0,
           W1, root1, bias1, gamma1, beta1, finW, finb):
    f32 = jnp.float32
    src = edge_index[0]
    dst = edge_index[1]

    a0 = edge_attr[:, 0].reshape(E // C, C)
    a1 = edge_attr[:, 1].reshape(E // C, C)
    f0, f1, base = _edge_prep(a0, a1)
    f0 = f0.reshape(E)
    f1 = f1.reshape(E)
    base = base.reshape(E)

    ones_w = jnp.ones((NW, C), f32)
    ones_1 = jnp.ones((1, C), f32)
    xbits = jnp.sum(x.reshape(N // 16, 16) << jnp.arange(16, dtype=jnp.int32),
                    axis=1).astype(jnp.int32)
    xb2 = jnp.broadcast_to(xbits[:, None], (N // 16, 16)).reshape(N)

    w0t = W0.transpose(1, 0, 2).reshape(C, KTOT * C)
    p0 = _matmul(emb, w0t).reshape(2 * KTOT, C)

    degp, gfull = _deg_kernel(dst, src, base, xb2)
    degp = degp.reshape(NW, N)
    msgp0 = _l0_kernel(dst, f0, f1, gfull, p0)
    h1 = _l0_dense(msgp0, degp, x.astype(f32).reshape(1, N), emb, root0,
                   bias0.reshape(1, C), gamma0.reshape(1, C),
                   beta0.reshape(1, C), ones_w, ones_1)

    w1t = W1.transpose(1, 0, 2).reshape(C, KTOT * C)
    p1 = _p1_matmul(h1, w1t).reshape(N * KTOT, C)

    msgp1 = _l1_kernel(src, dst, f0, f1, base, p1)
    out = _l1_dense(msgp1, degp, h1, root1, bias1.reshape(1, C),
                    gamma1.reshape(1, C), beta1.reshape(1, C), finW,
                    finb.reshape(1, C), ones_w)
    return out
